# baseline (speedup 1.0000x reference)
import functools

import jax
import jax.numpy as jnp
from jax import lax
from jax.experimental import pallas as pl
from jax.experimental.pallas import tpu as pltpu
from jax.experimental.pallas import tpu_sc as plsc

B = 16384
L = 200
DIM = 64
NC, NS = 2, 16
NW = NC * NS
RPW = B // NW
GR = 16
NGRP = RPW // GR
WPAD = 208
CS = (64, 64, 64, 8)
CO = (0, 64, 128, 192)


def _sc_body(x_hbm, len_hbm, emb_hbm, wt_hbm, out_hbm,
             idx_v, emb_v, w_v, len_v, out_v,
             isem, esem0, esem1, wsem0, wsem1, osem):
  esem = (esem0, esem1)
  wsem = (wsem0, wsem1)
  wid = lax.axis_index("s") * NC + lax.axis_index("c")
  base = wid * RPW

  pltpu.sync_copy(len_hbm.at[pl.ds(base, RPW)], len_v)
  pltpu.async_copy(x_hbm.at[pl.ds(base * L, GR * L)], idx_v.at[0], isem)

  zvec = jnp.zeros((16,), jnp.float32)

  @pl.loop(0, WPAD)
  def _zinit(t):
    for ebi in range(2):
      for k in range(4):
        emb_v[ebi, t, pl.ds(16 * k, 16)] = zvec

  def fetch(db, j, eb, lnv):
    nch = (lnv[j] + 63) // 64
    for c in range(4):
      @pl.when(c < nch)
      def _(c=c):
        pltpu.async_copy(emb_hbm.at[idx_v.at[db, pl.ds(j * L + CO[c], CS[c])]],
                         emb_v.at[eb, pl.ds(CO[c], CS[c])], esem[eb])
        pltpu.async_copy(wt_hbm.at[idx_v.at[db, pl.ds(j * L + CO[c], CS[c])]],
                         w_v.at[eb, pl.ds(CO[c], CS[c])], wsem[eb])

  def wait_chunk(db, j, eb, c):
    pltpu.make_async_copy(emb_hbm.at[idx_v.at[db, pl.ds(j * L + CO[c], CS[c])]],
                          emb_v.at[eb, pl.ds(CO[c], CS[c])], esem[eb]).wait()
    pltpu.make_async_copy(wt_hbm.at[idx_v.at[db, pl.ds(j * L + CO[c], CS[c])]],
                          w_v.at[eb, pl.ds(CO[c], CS[c])], wsem[eb]).wait()

  @pl.loop(0, NGRP)
  def _group(g):
    db = lax.rem(g, 2)
    rowbase = base + g * GR
    pltpu.make_async_copy(x_hbm.at[pl.ds(rowbase * L, GR * L)], idx_v.at[db],
                          isem).wait()

    @pl.when(g + 1 < NGRP)
    def _():
      pltpu.async_copy(x_hbm.at[pl.ds((rowbase + GR) * L, GR * L)],
                       idx_v.at[1 - db], isem)

    lnv = len_v[pl.ds(g * GR, GR)]
    fetch(db, 0, 0, lnv)
    for j in range(GR):
      eb = j % 2
      if j + 1 < GR:
        fetch(db, j + 1, 1 - eb, lnv)

      ln = lnv[j]
      nch = (ln + 63) // 64
      nmg = (ln + 15) // 16
      zero = jnp.zeros((16,), jnp.float32)

      def grp(m, acc, eb=eb, ln=ln):
        wg = w_v[eb, pl.ds(m * 16, 16)]
        pos = lax.iota(jnp.int32, 16) + m * 16
        wgm = jnp.where(pos < ln, wg, 0.0)
        for t in range(16):
          wt = wgm[t]
          acc = tuple(
              acc[k] + wt * emb_v[eb, m * 16 + t, pl.ds(16 * k, 16)]
              for k in range(4))
        return acc

      for c in range(4):
        @pl.when(c < nch)
        def _(c=c):
          wait_chunk(db, j, eb, c)
      acc = lax.fori_loop(0, nmg, grp, (zero, zero, zero, zero))
      for k in range(4):
        out_v[db, j, pl.ds(16 * k, 16)] = acc[k]

    @pl.when(g > 0)
    def _():
      pltpu.make_async_copy(out_v.at[1 - db],
                            out_hbm.at[pl.ds(rowbase - GR, GR)], osem).wait()

    pltpu.async_copy(out_v.at[db], out_hbm.at[pl.ds(rowbase, GR)], osem)

  pltpu.make_async_copy(out_v.at[(NGRP - 1) % 2],
                        out_hbm.at[pl.ds(base + (NGRP - 1) * GR, GR)],
                        osem).wait()


def _sc_weighted_sums(x, lengths, emb_table, wt_flat):
  mesh = plsc.VectorSubcoreMesh(core_axis_name="c", subcore_axis_name="s",
                                num_cores=NC, num_subcores=NS)
  f = pl.kernel(
      _sc_body,
      out_type=jax.ShapeDtypeStruct((B, DIM), jnp.float32),
      mesh=mesh,
      compiler_params=pltpu.CompilerParams(use_tc_tiling_on_sc=False),
      scratch_types=[
          pltpu.VMEM((2, GR * L), jnp.int32),
          pltpu.VMEM((2, WPAD, 2 * DIM), jnp.float32),
          pltpu.VMEM((2, WPAD), jnp.float32),
          pltpu.VMEM((RPW,), jnp.int32),
          pltpu.VMEM((2, GR, DIM), jnp.float32),
          pltpu.SemaphoreType.DMA,
          pltpu.SemaphoreType.DMA,
          pltpu.SemaphoreType.DMA,
          pltpu.SemaphoreType.DMA,
          pltpu.SemaphoreType.DMA,
          pltpu.SemaphoreType.DMA,
      ],
  )
  return f(x.reshape(-1), lengths, emb_table, wt_flat)


def _tc_body(s_ref, l_ref, w_ref, b_ref, o_ref):
  avg = s_ref[...] / l_ref[...].astype(jnp.float32)
  o_ref[...] = (
      jnp.dot(avg, w_ref[...], preferred_element_type=jnp.float32)
      + b_ref[...])


def _tc_project(summed, lengths, W_lin, b_lin):
  BLK = 2048
  return pl.pallas_call(
      _tc_body,
      grid=(B // BLK,),
      in_specs=[
          pl.BlockSpec((BLK, DIM), lambda i: (i, 0)),
          pl.BlockSpec((BLK, 1), lambda i: (i, 0)),
          pl.BlockSpec((DIM, DIM), lambda i: (0, 0)),
          pl.BlockSpec((1, DIM), lambda i: (0, 0)),
      ],
      out_specs=pl.BlockSpec((BLK, DIM), lambda i: (i, 0)),
      out_shape=jax.ShapeDtypeStruct((B, DIM), jnp.float32),
  )(summed, lengths.reshape(B, 1), W_lin, b_lin.reshape(1, DIM))


@jax.jit
def kernel(x, lengths, emb_table, weight_table, W_lin, b_lin):
  wt_flat = weight_table.reshape(-1)
  emb128 = jnp.pad(emb_table, ((0, 0), (0, DIM)))
  summed = _sc_weighted_sums(x, lengths, emb128, wt_flat)
  return _tc_project(summed, lengths, W_lin, b_lin)


# speedup vs baseline: 1.1852x; 1.1852x over previous
import functools

import jax
import jax.numpy as jnp
from jax import lax
from jax.experimental import pallas as pl
from jax.experimental.pallas import tpu as pltpu
from jax.experimental.pallas import tpu_sc as plsc

B = 16384
L = 200
DIM = 64
NC, NS = 2, 16
NW = NC * NS
RPW = B // NW
GR = 16
NGRP = RPW // GR
WPAD = 208
CS = (64, 64, 64, 8)
CO = (0, 64, 128, 192)


def _sc_body(x_hbm, len_hbm, emb_hbm, wt_hbm, out_hbm,
             idx_v, emb_v, w_v, len_v, out_v,
             isem, esem0, esem1, esem2, esem3,
             wsem0, wsem1, wsem2, wsem3, osem):
  esem = (esem0, esem1, esem2, esem3)
  wsem = (wsem0, wsem1, wsem2, wsem3)
  wid = lax.axis_index("s") * NC + lax.axis_index("c")
  base = wid * RPW

  pltpu.sync_copy(len_hbm.at[pl.ds(base, RPW)], len_v)
  pltpu.async_copy(x_hbm.at[pl.ds(base * L, GR * L)], idx_v.at[0], isem)

  zvec = jnp.zeros((16,), jnp.float32)

  @pl.loop(0, WPAD)
  def _zinit(t):
    for ebi in range(4):
      for k in range(4):
        emb_v[ebi, t, pl.ds(16 * k, 16)] = zvec

  def fetch(db, j, eb, lnv):
    nch = (lnv[j] + 63) // 64
    for c in range(4):
      @pl.when(c < nch)
      def _(c=c):
        pltpu.async_copy(emb_hbm.at[idx_v.at[db, pl.ds(j * L + CO[c], CS[c])]],
                         emb_v.at[eb, pl.ds(CO[c], CS[c])], esem[eb])
        pltpu.async_copy(wt_hbm.at[idx_v.at[db, pl.ds(j * L + CO[c], CS[c])]],
                         w_v.at[eb, pl.ds(CO[c], CS[c])], wsem[eb])

  def wait_chunk(db, j, eb, c):
    pltpu.make_async_copy(emb_hbm.at[idx_v.at[db, pl.ds(j * L + CO[c], CS[c])]],
                          emb_v.at[eb, pl.ds(CO[c], CS[c])], esem[eb]).wait()
    pltpu.make_async_copy(wt_hbm.at[idx_v.at[db, pl.ds(j * L + CO[c], CS[c])]],
                          w_v.at[eb, pl.ds(CO[c], CS[c])], wsem[eb]).wait()

  pltpu.make_async_copy(x_hbm.at[pl.ds(base * L, GR * L)], idx_v.at[0],
                        isem).wait()
  lnv0 = len_v[pl.ds(0, GR)]
  fetch(0, 0, 0, lnv0)
  fetch(0, 1, 1, lnv0)

  @pl.loop(0, NGRP)
  def _group(g):
    db = lax.rem(g, 2)
    rowbase = base + g * GR

    @pl.when(g + 1 < NGRP)
    def _():
      pltpu.async_copy(x_hbm.at[pl.ds((rowbase + GR) * L, GR * L)],
                       idx_v.at[1 - db], isem)

    lnv = len_v[pl.ds(g * GR, GR)]
    lnv_n = len_v[pl.ds(jnp.minimum(g + 1, NGRP - 1) * GR, GR)]
    for j in range(GR):
      eb = j % 4
      if j + 2 < GR:
        fetch(db, j + 2, (j + 2) % 4, lnv)
      if j == GR - 3:
        @pl.when(g + 1 < NGRP)
        def _():
          pltpu.make_async_copy(
              x_hbm.at[pl.ds((rowbase + GR) * L, GR * L)],
              idx_v.at[1 - db], isem).wait()
      if j >= GR - 2:
        @pl.when(g + 1 < NGRP)
        def _(j=j):
          fetch(1 - db, j - (GR - 2), (j - (GR - 2)) % 4, lnv_n)

      ln = lnv[j]
      nch = (ln + 63) // 64
      nmg = (ln + 15) // 16
      zero = jnp.zeros((16,), jnp.float32)

      def grp(m, acc, eb=eb, ln=ln):
        wg = w_v[eb, pl.ds(m * 16, 16)]
        pos = lax.iota(jnp.int32, 16) + m * 16
        wgm = jnp.where(pos < ln, wg, 0.0)
        for t in range(16):
          wt = wgm[t]
          acc = tuple(
              acc[k] + wt * emb_v[eb, m * 16 + t, pl.ds(16 * k, 16)]
              for k in range(4))
        return acc

      for c in range(4):
        @pl.when(c < nch)
        def _(c=c):
          wait_chunk(db, j, eb, c)
      acc = lax.fori_loop(0, nmg, grp, (zero, zero, zero, zero))
      for k in range(4):
        out_v[db, j, pl.ds(16 * k, 16)] = acc[k]

    @pl.when(g > 0)
    def _():
      pltpu.make_async_copy(out_v.at[1 - db],
                            out_hbm.at[pl.ds(rowbase - GR, GR)], osem).wait()

    pltpu.async_copy(out_v.at[db], out_hbm.at[pl.ds(rowbase, GR)], osem)

  pltpu.make_async_copy(out_v.at[(NGRP - 1) % 2],
                        out_hbm.at[pl.ds(base + (NGRP - 1) * GR, GR)],
                        osem).wait()


def _sc_weighted_sums(x, lengths, emb_table, wt_flat):
  mesh = plsc.VectorSubcoreMesh(core_axis_name="c", subcore_axis_name="s",
                                num_cores=NC, num_subcores=NS)
  f = pl.kernel(
      _sc_body,
      out_type=jax.ShapeDtypeStruct((B, DIM), jnp.float32),
      mesh=mesh,
      compiler_params=pltpu.CompilerParams(use_tc_tiling_on_sc=False),
      scratch_types=[
          pltpu.VMEM((2, GR * L), jnp.int32),
          pltpu.VMEM((4, WPAD, DIM), jnp.float32),
          pltpu.VMEM((4, WPAD), jnp.float32),
          pltpu.VMEM((RPW,), jnp.int32),
          pltpu.VMEM((2, GR, DIM), jnp.float32),
          pltpu.SemaphoreType.DMA,
          pltpu.SemaphoreType.DMA,
          pltpu.SemaphoreType.DMA,
          pltpu.SemaphoreType.DMA,
          pltpu.SemaphoreType.DMA,
          pltpu.SemaphoreType.DMA,
          pltpu.SemaphoreType.DMA,
          pltpu.SemaphoreType.DMA,
          pltpu.SemaphoreType.DMA,
          pltpu.SemaphoreType.DMA,
      ],
  )
  return f(x.reshape(-1), lengths, emb_table, wt_flat)


def _tc_body(s_ref, l_ref, w_ref, b_ref, o_ref):
  avg = s_ref[...] / l_ref[...].astype(jnp.float32)
  o_ref[...] = (
      jnp.dot(avg, w_ref[...], preferred_element_type=jnp.float32)
      + b_ref[...])


def _tc_project(summed, lengths, W_lin, b_lin):
  BLK = 2048
  return pl.pallas_call(
      _tc_body,
      grid=(B // BLK,),
      in_specs=[
          pl.BlockSpec((BLK, DIM), lambda i: (i, 0)),
          pl.BlockSpec((BLK, 1), lambda i: (i, 0)),
          pl.BlockSpec((DIM, DIM), lambda i: (0, 0)),
          pl.BlockSpec((1, DIM), lambda i: (0, 0)),
      ],
      out_specs=pl.BlockSpec((BLK, DIM), lambda i: (i, 0)),
      out_shape=jax.ShapeDtypeStruct((B, DIM), jnp.float32),
  )(summed, lengths.reshape(B, 1), W_lin, b_lin.reshape(1, DIM))


@jax.jit
def kernel(x, lengths, emb_table, weight_table, W_lin, b_lin):
  wt_flat = weight_table.reshape(-1)
  summed = _sc_weighted_sums(x, lengths, emb_table, wt_flat)
  return _tc_project(summed, lengths, W_lin, b_lin)
